# Initial kernel scaffold; baseline (speedup 1.0000x reference)
import jax
import jax.numpy as jnp
from jax.experimental import pallas as pl


def kernel(text, code, W_text, W_code):
    raise NotImplementedError("write your pallas kernel here")


# speedup vs baseline: 1.9611x; 1.9611x over previous
import functools

import jax
import jax.numpy as jnp
from jax import lax
from jax.experimental import pallas as pl
from jax.experimental.pallas import tpu as pltpu
from jax.experimental.pallas import tpu_sc as plsc

LANES = 16
NUM_WORKERS = 32


def _make_sc_kernel(B, L, Lp, D, V):
    assert B % NUM_WORKERS == 0 and D % LANES == 0 and Lp % 8 == 0
    ipw = B // NUM_WORKERS
    nch = D // LANES
    inv_l2 = 1.0 / float(L * L)

    mesh = plsc.VectorSubcoreMesh(core_axis_name="c", subcore_axis_name="s")

    @functools.partial(
        pl.kernel,
        out_type=jax.ShapeDtypeStruct((B,), jnp.float32),
        mesh=mesh,
        compiler_params=pltpu.CompilerParams(needs_layout_passes=False),
        scratch_types=[
            pltpu.VMEM((ipw, Lp), jnp.int32),
            pltpu.VMEM((ipw, Lp), jnp.int32),
            pltpu.VMEM((Lp, D), jnp.float32),
            pltpu.VMEM((Lp, D), jnp.float32),
            pltpu.VMEM((ipw * LANES,), jnp.float32),
            pltpu.VMEM((ipw,), jnp.float32),
            pltpu.SemaphoreType.DMA,
            pltpu.SemaphoreType.DMA,
        ],
    )
    def sc_kernel(text_hbm, code_hbm, wt_hbm, wc_hbm, out_hbm,
                  tidx, cidx, buf_t, buf_c, partials, outv, sem_t, sem_c):
        wid = lax.axis_index("s") * 2 + lax.axis_index("c")
        base = wid * ipw
        pltpu.sync_copy(text_hbm.at[pl.ds(base, ipw)], tidx)
        pltpu.sync_copy(code_hbm.at[pl.ds(base, ipw)], cidx)

        @pl.loop(0, ipw)
        def _item(i):
            cp_t = pltpu.async_copy(wt_hbm.at[tidx.at[i]], buf_t, sem_t)
            cp_c = pltpu.async_copy(wc_hbm.at[cidx.at[i]], buf_c, sem_c)
            cp_t.wait()
            cp_c.wait()

            def chunk_body(j, dot_acc):
                col = pl.ds(j * LANES, LANES)
                st = [buf_t[r, col] for r in range(4)]
                sc = [buf_c[r, col] for r in range(4)]
                for r in range(4, L):
                    st[r % 4] = st[r % 4] + buf_t[r, col]
                    sc[r % 4] = sc[r % 4] + buf_c[r, col]
                s_t = (st[0] + st[1]) + (st[2] + st[3])
                s_c = (sc[0] + sc[1]) + (sc[2] + sc[3])
                return dot_acc + s_t * s_c

            dot_acc = lax.fori_loop(
                0, nch, chunk_body, jnp.zeros((LANES,), jnp.float32))
            partials[pl.ds(i * LANES, LANES)] = dot_acc

        lane = lax.iota(jnp.int32, LANES)
        for g in range(ipw // LANES):
            row_base = g * (LANES * LANES) + lane * LANES
            acc = [plsc.load_gather(partials, [row_base + c]) for c in range(4)]
            for c in range(4, LANES):
                acc[c % 4] = acc[c % 4] + plsc.load_gather(
                    partials, [row_base + c])
            dots = (acc[0] + acc[1]) + (acc[2] + acc[3])
            outv[pl.ds(g * LANES, LANES)] = (
                1.0 / (1.0 + jnp.exp(-dots * inv_l2)))

        pltpu.sync_copy(outv, out_hbm.at[pl.ds(base, ipw)])

    return sc_kernel


def kernel(text, code, W_text, W_code):
    B, L = text.shape
    V, D = W_text.shape
    Lp = (L + 7) // 8 * 8
    text = text.astype(jnp.int32)
    code = code.astype(jnp.int32)
    if Lp != L:
        pad = jnp.zeros((B, Lp - L), jnp.int32)
        text = jnp.concatenate([text, pad], axis=1)
        code = jnp.concatenate([code, pad], axis=1)
    fn = _make_sc_kernel(B, L, Lp, D, V)
    return fn(text, code, W_text, W_code)
